# baseline (speedup 1.0000x reference)
import functools

import jax
import jax.numpy as jnp
from jax import lax
from jax.experimental import pallas as pl
from jax.experimental.pallas import tpu as pltpu
from jax.experimental.pallas import tpu_sc as plsc

VOCAB = 1000000
EMBED = 64
BATCH = 16384

NC = 2
NS = 16
L = 16
NW = NC * NS
B_PER_W = BATCH // NW
RP = 64
ROUNDS = B_PER_W // RP
GPR = RP // L


def _w2v_dots(blk_hbm, off_hbm, emb2_hbm, out_hbm,
              blk_v, off_v, tbuf, cbuf, out_v, sem0, sem1):
    wid = lax.axis_index("s") * NC + lax.axis_index("c")
    base = wid * B_PER_W
    sems = (sem0, sem1)
    iota = lax.iota(jnp.int32, L)

    pltpu.sync_copy(blk_hbm.at[wid], blk_v)
    pltpu.sync_copy(off_hbm.at[wid], off_v)

    def fire(r, buf):
        row = r >> 1
        col = (r & 1) * RP
        pltpu.async_copy(
            emb2_hbm.at[blk_v.at[row, pl.ds(col, RP)]], tbuf.at[buf], sems[buf])
        pltpu.async_copy(
            emb2_hbm.at[blk_v.at[4 + row, pl.ds(col, RP)]], cbuf.at[buf],
            sems[buf])

    def wait(buf):
        dummy = emb2_hbm.at[pl.ds(0, RP)]
        pltpu.make_async_copy(dummy, tbuf.at[buf], sems[buf]).wait()
        pltpu.make_async_copy(dummy, cbuf.at[buf], sems[buf]).wait()

    def compute(r, buf):
        bufv = jnp.full((L,), buf, jnp.int32)
        row = r >> 1

        def group(g, carry):
            col = (r & 1) * RP + g * L
            toff = off_v[row, pl.ds(col, L)]
            coff = off_v[4 + row, pl.ds(col, L)]
            slot = g * L + iota
            acc = jnp.zeros((L,), jnp.float32)
            for e in range(EMBED):
                t = plsc.load_gather(tbuf, [bufv, slot, toff + e])
                c = plsc.load_gather(cbuf, [bufv, slot, coff + e])
                acc = acc + t * c
            out_v[pl.ds(r * RP + g * L, L)] = acc
            return carry

        lax.fori_loop(0, GPR, group, 0)

    fire(0, 0)

    def body(i, carry):
        for half in range(2):
            r = 2 * i + half
            wait(half)
            if half == 0:
                fire(r + 1, 1)
            else:
                @pl.when(i < ROUNDS // 2 - 1)
                def _():
                    fire(r + 1, 0)
            compute(r, half)
        return carry

    lax.fori_loop(0, ROUNDS // 2, body, 0)

    pltpu.sync_copy(out_v, out_hbm.at[pl.ds(base, B_PER_W)])


@functools.cache
def _build():
    return pl.kernel(
        _w2v_dots,
        mesh=plsc.VectorSubcoreMesh(core_axis_name="c", subcore_axis_name="s"),
        compiler_params=pltpu.CompilerParams(needs_layout_passes=False),
        out_type=jax.ShapeDtypeStruct((BATCH,), jnp.float32),
        scratch_types=[
            pltpu.VMEM((8, 128), jnp.int32),
            pltpu.VMEM((8, 128), jnp.int32),
            pltpu.VMEM((2, RP, 128), jnp.float32),
            pltpu.VMEM((2, RP, 128), jnp.float32),
            pltpu.VMEM((B_PER_W,), jnp.float32),
            pltpu.SemaphoreType.DMA,
            pltpu.SemaphoreType.DMA,
        ],
    )


def kernel(xb, embed):
    idx = xb.astype(jnp.int32).reshape(2, NW, 4, 128)
    idx = jnp.concatenate([idx[0], idx[1]], axis=1)
    blk = idx >> 1
    off = (idx & 1) * EMBED
    return _build()(blk, off, embed.reshape(VOCAB // 2, 2 * EMBED))


# speedup vs baseline: 1.0043x; 1.0043x over previous
import functools

import jax
import jax.numpy as jnp
from jax import lax
from jax.experimental import pallas as pl
from jax.experimental.pallas import tpu as pltpu
from jax.experimental.pallas import tpu_sc as plsc

VOCAB = 1000000
EMBED = 64
BATCH = 16384

NC = 2
NS = 16
L = 16
NW = NC * NS
B_PER_W = BATCH // NW
RP = 64
ROUNDS = B_PER_W // RP
GPR = RP // L


def _w2v_dots(blk_hbm, off_hbm, emb2_hbm, out_hbm,
              blk_v, off_v, tbuf, cbuf, out_v, sem0, sem1):
    wid = lax.axis_index("s") * NC + lax.axis_index("c")
    base = wid * B_PER_W
    sems = (sem0, sem1)
    iota = lax.iota(jnp.int32, L)

    pltpu.sync_copy(blk_hbm.at[wid], blk_v)
    pltpu.sync_copy(off_hbm.at[wid], off_v)

    def fire(r, buf):
        row = r >> 1
        col = (r & 1) * RP
        pltpu.async_copy(
            emb2_hbm.at[blk_v.at[row, pl.ds(col, RP)]], tbuf.at[buf], sems[buf])
        pltpu.async_copy(
            emb2_hbm.at[blk_v.at[4 + row, pl.ds(col, RP)]], cbuf.at[buf],
            sems[buf])

    def wait(buf):
        dummy = emb2_hbm.at[pl.ds(0, RP)]
        pltpu.make_async_copy(dummy, tbuf.at[buf], sems[buf]).wait()
        pltpu.make_async_copy(dummy, cbuf.at[buf], sems[buf]).wait()

    def compute(r, buf):
        bufv = jnp.full((L,), buf, jnp.int32)
        row = r >> 1

        def group(g, carry):
            col = (r & 1) * RP + g * L
            toff = off_v[row, pl.ds(col, L)]
            coff = off_v[4 + row, pl.ds(col, L)]
            slot = g * L + iota
            acc = jnp.zeros((L,), jnp.float32)
            for e in range(EMBED):
                t = plsc.load_gather(tbuf, [bufv, slot, toff + e])
                c = plsc.load_gather(cbuf, [bufv, slot, coff + e])
                acc = acc + t * c
            out_v[pl.ds(r * RP + g * L, L)] = acc
            return carry

        lax.fori_loop(0, GPR, group, 0)

    fire(0, 0)

    def body(i, carry):
        for half in range(2):
            r = 2 * i + half
            wait(half)
            if half == 0:
                fire(r + 1, 1)
            else:
                @pl.when(i < ROUNDS // 2 - 1)
                def _():
                    fire(r + 1, 0)
            compute(r, half)
        return carry

    lax.fori_loop(0, ROUNDS // 2, body, 0)

    pltpu.sync_copy(out_v, out_hbm.at[pl.ds(base, B_PER_W)])


@functools.cache
def _build():
    return pl.kernel(
        _w2v_dots,
        mesh=plsc.VectorSubcoreMesh(core_axis_name="c", subcore_axis_name="s"),
        compiler_params=pltpu.CompilerParams(
            needs_layout_passes=False, use_tc_tiling_on_sc=False),
        out_type=jax.ShapeDtypeStruct((BATCH,), jnp.float32),
        scratch_types=[
            pltpu.VMEM((8, 128), jnp.int32),
            pltpu.VMEM((8, 128), jnp.int32),
            pltpu.VMEM((2, RP, 128), jnp.float32),
            pltpu.VMEM((2, RP, 128), jnp.float32),
            pltpu.VMEM((B_PER_W,), jnp.float32),
            pltpu.SemaphoreType.DMA,
            pltpu.SemaphoreType.DMA,
        ],
    )


def kernel(xb, embed):
    idx = xb.astype(jnp.int32).reshape(2, NW, 4, 128)
    idx = jnp.concatenate([idx[0], idx[1]], axis=1)
    blk = idx >> 1
    off = (idx & 1) * EMBED
    return _build()(blk, off, embed.reshape(VOCAB // 2, 2 * EMBED))


# speedup vs baseline: 2.1768x; 2.1674x over previous
import functools

import jax
import jax.numpy as jnp
from jax import lax
from jax.experimental import pallas as pl
from jax.experimental.pallas import tpu as pltpu
from jax.experimental.pallas import tpu_sc as plsc

VOCAB = 1000000
EMBED = 64
BATCH = 16384

NC = 2
NS = 16
L = 16
NW = NC * NS
B_PER_W = BATCH // NW
ROUNDS = B_PER_W // L


def _w2v_dots(idx_hbm, embed_hbm, out_hbm,
              idx_v, tbuf, cbuf, out_v, sem0, sem1):
    wid = lax.axis_index("s") * NC + lax.axis_index("c")
    base = wid * B_PER_W
    sems = (sem0, sem1)
    iota = lax.iota(jnp.int32, L)

    pltpu.sync_copy(idx_hbm.at[wid], idx_v)

    def round_idx(r, row_off):
        return idx_v[row_off + (r >> 3), pl.ds((r & 7) * L, L)]

    def fire(r, buf):
        tid = round_idx(r, 0) >> 3
        cid = round_idx(r, 4) >> 3
        for u in range(L):
            pltpu.async_copy(embed_hbm.at[tid[u]], tbuf.at[buf, u], sems[buf])
            pltpu.async_copy(embed_hbm.at[cid[u]], cbuf.at[buf, u], sems[buf])

    def wait(buf):
        dummy = embed_hbm.at[pl.ds(0, L)]
        pltpu.make_async_copy(dummy, tbuf.at[buf], sems[buf]).wait()
        pltpu.make_async_copy(dummy, cbuf.at[buf], sems[buf]).wait()

    def compute(r, buf):
        tsub = round_idx(r, 0) & 7
        csub = round_idx(r, 4) & 7
        bufv = jnp.full((L,), buf, jnp.int32)
        acc = jnp.zeros((L,), jnp.float32)
        for e in range(EMBED):
            ev = jnp.full((L,), e, jnp.int32)
            t = plsc.load_gather(tbuf, [bufv, iota, tsub, ev])
            c = plsc.load_gather(cbuf, [bufv, iota, csub, ev])
            acc = acc + t * c
        out_v[pl.ds(r * L, L)] = acc

    fire(0, 0)

    def body(i, carry):
        for half in range(2):
            r = 2 * i + half
            wait(half)
            if half == 0:
                fire(r + 1, 1)
            else:
                @pl.when(i < ROUNDS // 2 - 1)
                def _():
                    fire(r + 1, 0)
            compute(r, half)
        return carry

    lax.fori_loop(0, ROUNDS // 2, body, 0)

    pltpu.sync_copy(out_v, out_hbm.at[pl.ds(base, B_PER_W)])


@functools.cache
def _build():
    return pl.kernel(
        _w2v_dots,
        mesh=plsc.VectorSubcoreMesh(core_axis_name="c", subcore_axis_name="s"),
        compiler_params=pltpu.CompilerParams(needs_layout_passes=False),
        out_type=jax.ShapeDtypeStruct((BATCH,), jnp.float32),
        scratch_types=[
            pltpu.VMEM((8, 128), jnp.int32),
            pltpu.VMEM((2, L, 8, EMBED), jnp.float32),
            pltpu.VMEM((2, L, 8, EMBED), jnp.float32),
            pltpu.VMEM((B_PER_W,), jnp.float32),
            pltpu.SemaphoreType.DMA,
            pltpu.SemaphoreType.DMA,
        ],
    )


def kernel(xb, embed):
    idx = xb.astype(jnp.int32).reshape(2, NW, 4, 128)
    idx = jnp.concatenate([idx[0], idx[1]], axis=1)
    return _build()(idx, embed.reshape(VOCAB // 8, 8, EMBED))


# speedup vs baseline: 2.1778x; 1.0004x over previous
import functools

import jax
import jax.numpy as jnp
from jax import lax
from jax.experimental import pallas as pl
from jax.experimental.pallas import tpu as pltpu
from jax.experimental.pallas import tpu_sc as plsc

VOCAB = 1000000
EMBED = 64
BATCH = 16384

NC = 2
NS = 16
L = 16
NW = NC * NS
B_PER_W = BATCH // NW
ROUNDS = B_PER_W // L


def _w2v_dots(idx_hbm, embed_hbm, out_hbm,
              idx_v, tbuf, cbuf, out_v, sem0, sem1):
    wid = lax.axis_index("s") * NC + lax.axis_index("c")
    base = wid * B_PER_W
    sems = (sem0, sem1)
    iota = lax.iota(jnp.int32, L)

    pltpu.sync_copy(idx_hbm.at[wid], idx_v)

    def round_idx(r, row_off):
        return idx_v[row_off + (r >> 3), pl.ds((r & 7) * L, L)]

    def fire(r, buf):
        tid = round_idx(r, 0) >> 3
        cid = round_idx(r, 4) >> 3
        for u in range(L):
            pltpu.async_copy(embed_hbm.at[tid[u]], tbuf.at[buf, u], sems[buf])
            pltpu.async_copy(embed_hbm.at[cid[u]], cbuf.at[buf, u], sems[buf])

    def wait(buf):
        dummy = embed_hbm.at[pl.ds(0, L)]
        pltpu.make_async_copy(dummy, tbuf.at[buf], sems[buf]).wait()
        pltpu.make_async_copy(dummy, cbuf.at[buf], sems[buf]).wait()

    def compute(r, buf):
        tsub = round_idx(r, 0) & 7
        csub = round_idx(r, 4) & 7
        bufv = jnp.full((L,), buf, jnp.int32)
        acc = jnp.zeros((L,), jnp.float32)
        for e in range(EMBED):
            ev = jnp.full((L,), e, jnp.int32)
            t = plsc.load_gather(tbuf, [bufv, iota, tsub, ev])
            c = plsc.load_gather(cbuf, [bufv, iota, csub, ev])
            acc = acc + t * c
        out_v[pl.ds(r * L, L)] = acc

    fire(0, 0)

    def body(i, carry):
        for half in range(2):
            r = 2 * i + half
            wait(half)
            if half == 0:
                fire(r + 1, 1)
            else:
                @pl.when(i < ROUNDS // 2 - 1)
                def _():
                    fire(r + 1, 0)
            compute(r, half)
        return carry

    lax.fori_loop(0, ROUNDS // 2, body, 0)

    pltpu.sync_copy(out_v, out_hbm.at[pl.ds(base, B_PER_W)])


@functools.cache
def _build():
    return pl.kernel(
        _w2v_dots,
        mesh=plsc.VectorSubcoreMesh(core_axis_name="c", subcore_axis_name="s"),
        compiler_params=pltpu.CompilerParams(
            needs_layout_passes=False, skip_device_barrier=True),
        out_type=jax.ShapeDtypeStruct((BATCH,), jnp.float32),
        scratch_types=[
            pltpu.VMEM((8, 128), jnp.int32),
            pltpu.VMEM((2, L, 8, EMBED), jnp.float32),
            pltpu.VMEM((2, L, 8, EMBED), jnp.float32),
            pltpu.VMEM((B_PER_W,), jnp.float32),
            pltpu.SemaphoreType.DMA,
            pltpu.SemaphoreType.DMA,
        ],
    )


def kernel(xb, embed):
    idx = xb.astype(jnp.int32).reshape(2, NW, 4, 128)
    idx = jnp.concatenate([idx[0], idx[1]], axis=1)
    return _build()(idx, embed.reshape(VOCAB // 8, 8, EMBED))


# speedup vs baseline: 2.1786x; 1.0004x over previous
import functools

import jax
import jax.numpy as jnp
from jax import lax
from jax.experimental import pallas as pl
from jax.experimental.pallas import tpu as pltpu
from jax.experimental.pallas import tpu_sc as plsc

VOCAB = 1000000
EMBED = 64
BATCH = 16384

NC = 2
NS = 16
L = 16
NW = NC * NS
B_PER_W = BATCH // NW
ROUNDS = B_PER_W // L


def _w2v_dots(idx_hbm, embed_hbm, out_hbm,
              idx_v, tbuf, cbuf, out_v, sem0, sem1):
    wid = lax.axis_index("s") * NC + lax.axis_index("c")
    base = wid * B_PER_W
    sems = (sem0, sem1)
    iota = lax.iota(jnp.int32, L)

    pltpu.sync_copy(idx_hbm.at[wid], idx_v)

    def round_idx(r, row_off):
        return idx_v[row_off + (r >> 3), pl.ds((r & 7) * L, L)]

    def fire(r, buf):
        tid = round_idx(r, 0) >> 3
        cid = round_idx(r, 4) >> 3
        for u in range(L):
            pltpu.async_copy(embed_hbm.at[tid[u]], tbuf.at[buf, u], sems[buf])
            pltpu.async_copy(embed_hbm.at[cid[u]], cbuf.at[buf, u], sems[buf])

    def wait(buf):
        dummy = embed_hbm.at[pl.ds(0, L)]
        pltpu.make_async_copy(dummy, tbuf.at[buf], sems[buf]).wait()
        pltpu.make_async_copy(dummy, cbuf.at[buf], sems[buf]).wait()

    def compute(r, buf):
        tsub = round_idx(r, 0) & 7
        csub = round_idx(r, 4) & 7
        bufv = jnp.full((L,), buf, jnp.int32)
        acc = jnp.zeros((L,), jnp.float32)
        for e in range(EMBED):
            ev = jnp.full((L,), e, jnp.int32)
            t = plsc.load_gather(tbuf, [bufv, iota, tsub, ev])
            c = plsc.load_gather(cbuf, [bufv, iota, csub, ev])
            acc = acc + t * c
        out_v[pl.ds(r * L, L)] = acc

    fire(0, 0)

    def body(i, carry):
        for half in range(2):
            r = 2 * i + half
            wait(half)
            if half == 0:
                fire(r + 1, 1)
            else:
                @pl.when(i < ROUNDS // 2 - 1)
                def _():
                    fire(r + 1, 0)
            compute(r, half)
        return carry

    lax.fori_loop(0, ROUNDS // 2, body, 0)

    pltpu.sync_copy(out_v, out_hbm.at[pl.ds(base, B_PER_W)])


@functools.cache
def _build():
    return pl.kernel(
        _w2v_dots,
        mesh=plsc.VectorSubcoreMesh(core_axis_name="c", subcore_axis_name="s"),
        compiler_params=pltpu.CompilerParams(needs_layout_passes=False),
        out_type=jax.ShapeDtypeStruct((BATCH,), jnp.float32),
        scratch_types=[
            pltpu.VMEM((8, 128), jnp.int32),
            pltpu.VMEM((2, L, 8, EMBED), jnp.float32),
            pltpu.VMEM((2, L, 8, EMBED), jnp.float32),
            pltpu.VMEM((B_PER_W,), jnp.float32),
            pltpu.SemaphoreType.DMA,
            pltpu.SemaphoreType.DMA,
        ],
    )


def kernel(xb, embed):
    idx = xb.astype(jnp.int32).reshape(2, NW, 4, 128)
    idx = jnp.concatenate([idx[0], idx[1]], axis=1)
    return _build()(idx, embed.reshape(VOCAB // 8, 8, EMBED))
